# baseline (speedup 1.0000x reference)
import functools

import jax
import jax.numpy as jnp
from jax import lax
from jax.experimental import pallas as pl
from jax.experimental.pallas import tpu as pltpu
from jax.experimental.pallas import tpu_sc as plsc

EPS = 1e-12

NC = 2
NS = 16
LN = 16
CH = 128
NJ = 8


def _rsqrt_newton(v):
    i = plsc.bitcast(v, jnp.int32)
    y = plsc.bitcast(jnp.int32(0x5F3759DF) - lax.shift_right_arithmetic(i, 1),
                     jnp.float32)
    for _ in range(2):
        y = y * (1.5 - 0.5 * v * y * y)
    return y


def _tree_sum(vals):
    vals = list(vals)
    while len(vals) > 1:
        vals = [a + b for a, b in zip(vals[::2], vals[1::2])]
    return vals[0]


def _sc_body(per_w, n_chunks, x_hbm, tagf_hbm, const_hbm, out_hbm,
             xbuf0, xbuf1, obuf0, obuf1, tagbuf0, tagbuf1, cbuf,
             sx0, sx1, st0, st1, so0, so1):
    wid = lax.axis_index("s") * NC + lax.axis_index("c")
    base = wid * per_w
    pltpu.sync_copy(const_hbm, cbuf)
    t0v = [cbuf[pl.ds(j * LN, LN)] for j in range(NJ)]
    dfv = [cbuf[pl.ds(128 + j * LN, LN)] for j in range(NJ)]
    gv = [cbuf[pl.ds(256 + j * LN, LN)] for j in range(NJ)]
    bv = [cbuf[pl.ds(384 + j * LN, LN)] for j in range(NJ)]

    xbufs, obufs, tagbufs = (xbuf0, xbuf1), (obuf0, obuf1), (tagbuf0, tagbuf1)
    sxs, sts, sos = (sx0, sx1), (st0, st1), (so0, so1)

    def in_copies(ci, b):
        tok0 = base + (ci % n_chunks) * CH
        return (
            pltpu.make_async_copy(x_hbm.at[pl.ds(tok0, CH), :], xbufs[b], sxs[b]),
            pltpu.make_async_copy(tagf_hbm.at[pl.ds(tok0, CH)], tagbufs[b], sts[b]),
        )

    def out_copy(ci, b):
        tok0 = base + (ci % n_chunks) * CH
        return pltpu.make_async_copy(
            obufs[b], out_hbm.at[pl.ds(tok0, CH), :], sos[b])

    def compute(xbuf, tagbuf, obuf):
        def tok_body(tb):
            tagv = tagbuf[pl.ds(tb, LN)]
            for i in range(LN):
                t = tb + i
                tf = jnp.broadcast_to(tagv[i], (LN,))
                hs = []
                for j in range(NJ):
                    xj = xbuf[t, pl.ds(j * LN, LN)]
                    hs.append(xj + (t0v[j] + tf * dfv[j]))
                s = _tree_sum(hs)
                q = _tree_sum([h * h for h in hs])
                sumv = jnp.broadcast_to(jnp.sum(s), (LN,))
                sqv = jnp.broadcast_to(jnp.sum(q), (LN,))
                mean = sumv * (1.0 / 128.0)
                var = sqv * (1.0 / 128.0) - mean * mean
                rstd = _rsqrt_newton(var + EPS)
                for j in range(NJ):
                    obuf[t, pl.ds(j * LN, LN)] = (
                        (hs[j] - mean) * rstd * gv[j] + bv[j])

        plsc.parallel_loop(0, CH, LN, unroll=1)(tok_body)

    for c in in_copies(0, 0):
        c.start()

    def outer(ci0, _):
        for b in range(2):
            ci = 2 * ci0 + b
            for c in in_copies(ci + 1, 1 - b):
                c.start()
            for c in in_copies(ci, b):
                c.wait()
            @pl.when(ci >= 2)
            def _():
                out_copy(ci - 2, b).wait()
            compute(xbufs[b], tagbufs[b], obufs[b])
            out_copy(ci, b).start()
        return 0

    lax.fori_loop(0, n_chunks // 2, outer, 0)
    for c in in_copies(0, 0):
        c.wait()
    out_copy(n_chunks - 2, 0).wait()
    out_copy(n_chunks - 1, 1).wait()


def _sc_call(x2, tagf, consts):
    N, D = x2.shape
    NW = NC * NS
    per_w = N // NW
    n_chunks = per_w // CH
    body = functools.partial(_sc_body, per_w, n_chunks)
    f = pl.kernel(
        body,
        mesh=plsc.VectorSubcoreMesh(core_axis_name="c", subcore_axis_name="s"),
        compiler_params=pltpu.CompilerParams(needs_layout_passes=False),
        out_type=jax.ShapeDtypeStruct((N, D), jnp.float32),
        scratch_types=[
            pltpu.VMEM((CH, D), jnp.float32),
            pltpu.VMEM((CH, D), jnp.float32),
            pltpu.VMEM((CH, D), jnp.float32),
            pltpu.VMEM((CH, D), jnp.float32),
            pltpu.VMEM((CH,), jnp.float32),
            pltpu.VMEM((CH,), jnp.float32),
            pltpu.VMEM((4 * D,), jnp.float32),
            pltpu.SemaphoreType.DMA,
            pltpu.SemaphoreType.DMA,
            pltpu.SemaphoreType.DMA,
            pltpu.SemaphoreType.DMA,
            pltpu.SemaphoreType.DMA,
            pltpu.SemaphoreType.DMA,
        ],
    )
    return f(x2, tagf, consts)


def _tc_body(tags_ref, x_ref, emb_ref, gamma_ref, beta_ref, out_ref):
    x = x_ref[...]
    sel = tags_ref[...] != 0
    t0 = emb_ref[0, :][None, :]
    t1 = emb_ref[1, :][None, :]
    h = x + jnp.where(sel, t1, t0)
    mean = jnp.mean(h, axis=-1, keepdims=True)
    var = jnp.mean(jnp.square(h - mean), axis=-1, keepdims=True)
    rstd = lax.rsqrt(var + EPS)
    g = gamma_ref[0, :][None, :]
    b = beta_ref[0, :][None, :]
    out_ref[...] = (h - mean) * rstd * g + b


def _tc_call(x2, tagsc, emb_table, ln_gamma, ln_beta):
    N, D = x2.shape
    RB = 2048
    assert N % RB == 0
    return pl.pallas_call(
        _tc_body,
        grid=(N // RB,),
        in_specs=[
            pl.BlockSpec((RB, 1), lambda i: (i, 0)),
            pl.BlockSpec((RB, D), lambda i: (i, 0)),
            pl.BlockSpec((2, D), lambda i: (0, 0)),
            pl.BlockSpec((1, D), lambda i: (0, 0)),
            pl.BlockSpec((1, D), lambda i: (0, 0)),
        ],
        out_specs=pl.BlockSpec((RB, D), lambda i: (i, 0)),
        out_shape=jax.ShapeDtypeStruct((N, D), jnp.float32),
    )(tagsc, x2, emb_table, ln_gamma.reshape(1, D), ln_beta.reshape(1, D))


SC_ROWS = 335872


def kernel(x, emotion_tags, emb_table, ln_gamma, ln_beta):
    B, L, D = x.shape
    N = B * L
    assert D == 128
    x2 = x.reshape(N, D)
    tagf = emotion_tags.astype(jnp.float32).reshape(N)
    consts = jnp.concatenate(
        [emb_table[0], emb_table[1] - emb_table[0], ln_gamma, ln_beta])
    k = SC_ROWS
    tags2 = emotion_tags.astype(jnp.int32).reshape(N, 1)
    out_sc = _tc_call(x2[:k], tags2[:k], emb_table, ln_gamma, ln_beta)
    out_tc = _tc_call(x2[k:], tags2[k:], emb_table, ln_gamma, ln_beta)
    return jnp.concatenate([out_sc, out_tc], axis=0).reshape(B, L, D)


# speedup vs baseline: 1.3160x; 1.3160x over previous
import functools

import jax
import jax.numpy as jnp
from jax import lax
from jax.experimental import pallas as pl
from jax.experimental.pallas import tpu as pltpu
from jax.experimental.pallas import tpu_sc as plsc

EPS = 1e-12

NC = 2
NS = 16
LN = 16
CH = 128
NJ = 8


def _rsqrt_newton(v):
    i = plsc.bitcast(v, jnp.int32)
    y = plsc.bitcast(jnp.int32(0x5F3759DF) - lax.shift_right_arithmetic(i, 1),
                     jnp.float32)
    for _ in range(2):
        y = y * (1.5 - 0.5 * v * y * y)
    return y


def _tree_sum(vals):
    vals = list(vals)
    while len(vals) > 1:
        vals = [a + b for a, b in zip(vals[::2], vals[1::2])]
    return vals[0]


def _sc_body(per_w, n_chunks, x_hbm, tagf_hbm, const_hbm, out_hbm,
             xbuf0, xbuf1, obuf0, obuf1, tagbuf0, tagbuf1, cbuf, sums, sqs,
             sx0, sx1, st0, st1, so0, so1):
    wid = lax.axis_index("s") * NC + lax.axis_index("c")
    base = wid * per_w
    pltpu.sync_copy(const_hbm, cbuf)
    t0v = [cbuf[pl.ds(j * LN, LN)] for j in range(NJ)]
    dfv = [cbuf[pl.ds(128 + j * LN, LN)] for j in range(NJ)]
    gv = [cbuf[pl.ds(256 + j * LN, LN)] for j in range(NJ)]
    bv = [cbuf[pl.ds(384 + j * LN, LN)] for j in range(NJ)]

    xbufs, obufs, tagbufs = (xbuf0, xbuf1), (obuf0, obuf1), (tagbuf0, tagbuf1)
    sxs, sts, sos = (sx0, sx1), (st0, st1), (so0, so1)

    def in_copies(ci, b):
        tok0 = base + (ci % n_chunks) * CH
        return (
            pltpu.make_async_copy(x_hbm.at[pl.ds(tok0, CH), :], xbufs[b], sxs[b]),
            pltpu.make_async_copy(tagf_hbm.at[pl.ds(tok0, CH)], tagbufs[b], sts[b]),
        )

    def out_copy(ci, b):
        tok0 = base + (ci % n_chunks) * CH
        return pltpu.make_async_copy(
            obufs[b], out_hbm.at[pl.ds(tok0, CH), :], sos[b])

    def compute(xbuf, tagbuf, obuf, sums, sqs):
        def tok_body(tb):
            tagv = tagbuf[pl.ds(tb, LN)]
            for i in range(LN):
                t = tb + i
                tf = jnp.broadcast_to(tagv[i], (LN,))
                hs = []
                for j in range(NJ):
                    xj = xbuf[t, pl.ds(j * LN, LN)]
                    hj = xj + (t0v[j] + tf * dfv[j])
                    obuf[t, pl.ds(j * LN, LN)] = hj
                    hs.append(hj)
                s = _tree_sum(hs)
                q = _tree_sum([h * h for h in hs])
                sums[t, :] = jnp.broadcast_to(jnp.sum(s), (LN,))
                sqs[t, :] = jnp.broadcast_to(jnp.sum(q), (LN,))
            lanes = lax.iota(jnp.int32, LN)
            zcol = jnp.zeros((LN,), jnp.int32)
            sv = plsc.load_gather(sums, [tb + lanes, zcol])
            qv = plsc.load_gather(sqs, [tb + lanes, zcol])
            meanv = sv * (1.0 / 128.0)
            varv = qv * (1.0 / 128.0) - meanv * meanv
            rstdv = _rsqrt_newton(varv + EPS)
            for i in range(LN):
                t = tb + i
                mean = jnp.broadcast_to(meanv[i], (LN,))
                rstd = jnp.broadcast_to(rstdv[i], (LN,))
                for j in range(NJ):
                    hj = obuf[t, pl.ds(j * LN, LN)]
                    obuf[t, pl.ds(j * LN, LN)] = (
                        (hj - mean) * rstd * gv[j] + bv[j])

        plsc.parallel_loop(0, CH, LN, unroll=1)(tok_body)

    for c in in_copies(0, 0):
        c.start()

    def outer(ci0, _):
        for b in range(2):
            ci = 2 * ci0 + b
            for c in in_copies(ci + 1, 1 - b):
                c.start()
            for c in in_copies(ci, b):
                c.wait()
            @pl.when(ci >= 2)
            def _():
                out_copy(ci - 2, b).wait()
            compute(xbufs[b], tagbufs[b], obufs[b], sums, sqs)
            out_copy(ci, b).start()
        return 0

    lax.fori_loop(0, n_chunks // 2, outer, 0)
    for c in in_copies(0, 0):
        c.wait()
    out_copy(n_chunks - 2, 0).wait()
    out_copy(n_chunks - 1, 1).wait()


def _sc_call(x2, tagf, consts):
    N, D = x2.shape
    NW = NC * NS
    per_w = N // NW
    n_chunks = per_w // CH
    body = functools.partial(_sc_body, per_w, n_chunks)
    f = pl.kernel(
        body,
        mesh=plsc.VectorSubcoreMesh(core_axis_name="c", subcore_axis_name="s"),
        compiler_params=pltpu.CompilerParams(needs_layout_passes=False),
        out_type=jax.ShapeDtypeStruct((N, D), jnp.float32),
        scratch_types=[
            pltpu.VMEM((CH, D), jnp.float32),
            pltpu.VMEM((CH, D), jnp.float32),
            pltpu.VMEM((CH, D), jnp.float32),
            pltpu.VMEM((CH, D), jnp.float32),
            pltpu.VMEM((CH,), jnp.float32),
            pltpu.VMEM((CH,), jnp.float32),
            pltpu.VMEM((4 * D,), jnp.float32),
            pltpu.VMEM((CH, LN), jnp.float32),
            pltpu.VMEM((CH, LN), jnp.float32),
            pltpu.SemaphoreType.DMA,
            pltpu.SemaphoreType.DMA,
            pltpu.SemaphoreType.DMA,
            pltpu.SemaphoreType.DMA,
            pltpu.SemaphoreType.DMA,
            pltpu.SemaphoreType.DMA,
        ],
    )
    return f(x2, tagf, consts)


def _tc_body(tags_ref, x_ref, emb_ref, gamma_ref, beta_ref, out_ref):
    x = x_ref[...]
    sel = tags_ref[...] != 0
    t0 = emb_ref[0, :][None, :]
    t1 = emb_ref[1, :][None, :]
    h = x + jnp.where(sel, t1, t0)
    mean = jnp.mean(h, axis=-1, keepdims=True)
    var = jnp.mean(jnp.square(h - mean), axis=-1, keepdims=True)
    rstd = lax.rsqrt(var + EPS)
    g = gamma_ref[0, :][None, :]
    b = beta_ref[0, :][None, :]
    out_ref[...] = (h - mean) * rstd * g + b


def _tc_call(x2, tagsc, emb_table, ln_gamma, ln_beta):
    N, D = x2.shape
    RB = 2048
    assert N % RB == 0
    return pl.pallas_call(
        _tc_body,
        grid=(N // RB,),
        in_specs=[
            pl.BlockSpec((RB, 1), lambda i: (i, 0)),
            pl.BlockSpec((RB, D), lambda i: (i, 0)),
            pl.BlockSpec((2, D), lambda i: (0, 0)),
            pl.BlockSpec((1, D), lambda i: (0, 0)),
            pl.BlockSpec((1, D), lambda i: (0, 0)),
        ],
        out_specs=pl.BlockSpec((RB, D), lambda i: (i, 0)),
        out_shape=jax.ShapeDtypeStruct((N, D), jnp.float32),
    )(tagsc, x2, emb_table, ln_gamma.reshape(1, D), ln_beta.reshape(1, D))


SC_ROWS = 335872


def kernel(x, emotion_tags, emb_table, ln_gamma, ln_beta):
    B, L, D = x.shape
    N = B * L
    assert D == 128
    x2 = x.reshape(N, D)
    tagf = emotion_tags.astype(jnp.float32).reshape(N)
    consts = jnp.concatenate(
        [emb_table[0], emb_table[1] - emb_table[0], ln_gamma, ln_beta])
    out = _sc_call(x2, tagf, consts)
    return out.reshape(B, L, D)


# speedup vs baseline: 1.6317x; 1.2398x over previous
import functools

import jax
import jax.numpy as jnp
from jax import lax
from jax.experimental import pallas as pl
from jax.experimental.pallas import tpu as pltpu
from jax.experimental.pallas import tpu_sc as plsc

EPS = 1e-12

NC = 2
NS = 16
LN = 16
CH = 128
NJ = 8


def _rsqrt_newton(v):
    i = plsc.bitcast(v, jnp.int32)
    y = plsc.bitcast(jnp.int32(0x5F3759DF) - lax.shift_right_arithmetic(i, 1),
                     jnp.float32)
    vh = 0.5 * v
    for _ in range(2):
        y = y * (1.5 - vh * y * y)
    return y


def _tree_sum(vals):
    vals = list(vals)
    while len(vals) > 1:
        vals = [a + b for a, b in zip(vals[::2], vals[1::2])]
    return vals[0]


def _sc_body(per_w, n_chunks, x_hbm, tagf_hbm, const_hbm, out_hbm,
             xbuf0, xbuf1, obuf0, obuf1, tagbuf0, tagbuf1, cbuf,
             sx0, sx1, st0, st1, so0, so1):
    wid = lax.axis_index("s") * NC + lax.axis_index("c")
    base = wid * per_w
    pltpu.sync_copy(const_hbm, cbuf)
    t0v = [cbuf[pl.ds(j * LN, LN)] for j in range(NJ)]
    dfv = [cbuf[pl.ds(128 + j * LN, LN)] for j in range(NJ)]
    gv = [cbuf[pl.ds(256 + j * LN, LN)] for j in range(NJ)]
    bv = [cbuf[pl.ds(384 + j * LN, LN)] for j in range(NJ)]

    xbufs, obufs, tagbufs = (xbuf0, xbuf1), (obuf0, obuf1), (tagbuf0, tagbuf1)
    sxs, sts, sos = (sx0, sx1), (st0, st1), (so0, so1)

    def in_copies(ci, b):
        tok0 = base + (ci % n_chunks) * CH
        return (
            pltpu.make_async_copy(x_hbm.at[pl.ds(tok0, CH), :], xbufs[b], sxs[b]),
            pltpu.make_async_copy(tagf_hbm.at[pl.ds(tok0, CH)], tagbufs[b], sts[b]),
        )

    def out_copy(ci, b):
        tok0 = base + (ci % n_chunks) * CH
        return pltpu.make_async_copy(
            obufs[b], out_hbm.at[pl.ds(tok0, CH), :], sos[b])

    def compute(xbuf, tagbuf, obuf):
        def tok_body(tb):
            tagv = tagbuf[pl.ds(tb, LN)]
            for i in range(LN):
                t = tb + i
                tf = jnp.broadcast_to(tagv[i], (LN,))
                hs = []
                for j in range(NJ):
                    xj = xbuf[t, pl.ds(j * LN, LN)]
                    hs.append(xj + (t0v[j] + tf * dfv[j]))
                s = _tree_sum(hs)
                q = _tree_sum([a * a + b * b for a, b in
                               zip(hs[::2], hs[1::2])])
                sumv = jnp.broadcast_to(jnp.sum(s), (LN,))
                sqv = jnp.broadcast_to(jnp.sum(q), (LN,))
                mean = sumv * (1.0 / 128.0)
                var = sqv * (1.0 / 128.0) - mean * mean
                rstd = _rsqrt_newton(var + EPS)
                for j in range(NJ):
                    obuf[t, pl.ds(j * LN, LN)] = (
                        (hs[j] - mean) * (rstd * gv[j]) + bv[j])

        plsc.parallel_loop(0, CH, LN, unroll=1)(tok_body)

    for c in in_copies(0, 0):
        c.start()

    def outer(ci0, _):
        for b in range(2):
            ci = 2 * ci0 + b
            for c in in_copies(ci + 1, 1 - b):
                c.start()
            for c in in_copies(ci, b):
                c.wait()
            @pl.when(ci >= 2)
            def _():
                out_copy(ci - 2, b).wait()
            compute(xbufs[b], tagbufs[b], obufs[b])
            out_copy(ci, b).start()
        return 0

    lax.fori_loop(0, n_chunks // 2, outer, 0)
    for c in in_copies(0, 0):
        c.wait()
    out_copy(n_chunks - 2, 0).wait()
    out_copy(n_chunks - 1, 1).wait()


def _sc_call(x2, tagf, consts):
    N, D = x2.shape
    NW = NC * NS
    per_w = N // NW
    n_chunks = per_w // CH
    body = functools.partial(_sc_body, per_w, n_chunks)
    f = pl.kernel(
        body,
        mesh=plsc.VectorSubcoreMesh(core_axis_name="c", subcore_axis_name="s"),
        compiler_params=pltpu.CompilerParams(needs_layout_passes=False),
        out_type=jax.ShapeDtypeStruct((N, D), jnp.float32),
        scratch_types=[
            pltpu.VMEM((CH, D), jnp.float32),
            pltpu.VMEM((CH, D), jnp.float32),
            pltpu.VMEM((CH, D), jnp.float32),
            pltpu.VMEM((CH, D), jnp.float32),
            pltpu.VMEM((CH,), jnp.float32),
            pltpu.VMEM((CH,), jnp.float32),
            pltpu.VMEM((4 * D,), jnp.float32),
            pltpu.SemaphoreType.DMA,
            pltpu.SemaphoreType.DMA,
            pltpu.SemaphoreType.DMA,
            pltpu.SemaphoreType.DMA,
            pltpu.SemaphoreType.DMA,
            pltpu.SemaphoreType.DMA,
        ],
    )
    return f(x2, tagf, consts)


def _tc_body(tags_ref, x_ref, emb_ref, gamma_ref, beta_ref, out_ref):
    x = x_ref[...]
    sel = tags_ref[...] != 0
    t0 = emb_ref[0, :][None, :]
    t1 = emb_ref[1, :][None, :]
    h = x + jnp.where(sel, t1, t0)
    mean = jnp.mean(h, axis=-1, keepdims=True)
    var = jnp.mean(jnp.square(h - mean), axis=-1, keepdims=True)
    rstd = lax.rsqrt(var + EPS)
    g = gamma_ref[0, :][None, :]
    b = beta_ref[0, :][None, :]
    out_ref[...] = (h - mean) * rstd * g + b


def _tc_call(x2, tagsc, emb_table, ln_gamma, ln_beta):
    N, D = x2.shape
    RB = 2048
    assert N % RB == 0
    return pl.pallas_call(
        _tc_body,
        grid=(N // RB,),
        in_specs=[
            pl.BlockSpec((RB, 1), lambda i: (i, 0)),
            pl.BlockSpec((RB, D), lambda i: (i, 0)),
            pl.BlockSpec((2, D), lambda i: (0, 0)),
            pl.BlockSpec((1, D), lambda i: (0, 0)),
            pl.BlockSpec((1, D), lambda i: (0, 0)),
        ],
        out_specs=pl.BlockSpec((RB, D), lambda i: (i, 0)),
        out_shape=jax.ShapeDtypeStruct((N, D), jnp.float32),
    )(tagsc, x2, emb_table, ln_gamma.reshape(1, D), ln_beta.reshape(1, D))


SC_ROWS = 335872


def kernel(x, emotion_tags, emb_table, ln_gamma, ln_beta):
    B, L, D = x.shape
    N = B * L
    assert D == 128
    x2 = x.reshape(N, D)
    tagf = emotion_tags.astype(jnp.float32).reshape(N)
    consts = jnp.concatenate(
        [emb_table[0], emb_table[1] - emb_table[0], ln_gamma, ln_beta])
    out = _sc_call(x2, tagf, consts)
    return out.reshape(B, L, D)
